# baseline (speedup 1.0000x reference)
import jax
import jax.numpy as jnp
from jax import lax
from jax.experimental import pallas as pl
from jax.experimental.pallas import tpu as pltpu
from jax.experimental.pallas import tpu_sc as plsc

N_E = 8192
E_DIM = 256
BETA = 0.25
N_TOK = 4608

T_BLK = 512
T_BLKS = N_TOK // T_BLK

R_TILE = 32
C_TILE = 256
N_RT = T_BLK // R_TILE
N_CT = N_E // C_TILE

SC_CORES = 2
SC_SUBCORES = 16
SC_WORKERS = SC_CORES * SC_SUBCORES
ROWS_PER_WORKER = N_TOK // SC_WORKERS


def _argmin_body(z_ref, cb_ref, zsq2_ref, esq2_ref, idx_ref, loss_ref,
                 oh_ref):
    t = pl.program_id(0)
    zb = z_ref[...]
    zs = zsq2_ref[...]

    vmin = None
    varg = None
    for c in range(N_CT):
        cb_c = cb_ref[c * C_TILE:(c + 1) * C_TILE, :]
        mm_c = lax.dot_general(zb, cb_c,
                               (((1,), (1,)), ((), ())))
        es = esq2_ref[:, c * C_TILE:(c + 1) * C_TILE]
        d_t = (zs + es) - mm_c
        if vmin is None:
            vmin = d_t
            varg = jnp.zeros((T_BLK, C_TILE), jnp.int32)
        else:
            better = d_t < vmin
            vmin = jnp.where(better, d_t, vmin)
            varg = jnp.where(better, c, varg)
    gidx = varg * C_TILE + lax.broadcasted_iota(jnp.int32, (T_BLK, C_TILE), 1)
    lmin = jnp.min(vmin, axis=1, keepdims=True)
    larg = jnp.min(jnp.where(vmin == lmin, gidx, N_E), axis=1,
                   keepdims=True)

    idx_ref[...] = larg
    iota_full = lax.broadcasted_iota(jnp.int32, (T_BLK, N_E), 1)
    oh_ref[...] = (larg == iota_full).astype(jnp.float32)

    @pl.when(t == 0)
    def _():
        loss_ref[0] = 0.0

    loss_ref[0] += jnp.sum(lmin)

    @pl.when(t == T_BLKS - 1)
    def _():
        loss_ref[0] = loss_ref[0] * (2.0 * (1.0 + BETA) / (N_TOK * E_DIM))


def _gather_body(cb_hbm, idx_hbm, zq_hbm, idx_v, rows_v, sem):
    wid = lax.axis_index("s") * SC_CORES + lax.axis_index("c")
    base = wid * ROWS_PER_WORKER
    pltpu.sync_copy(idx_hbm.at[pl.ds(base, ROWS_PER_WORKER)], idx_v)
    pltpu.async_copy(cb_hbm.at[idx_v], rows_v, sem).wait()
    pltpu.sync_copy(rows_v, zq_hbm.at[pl.ds(base, ROWS_PER_WORKER)])


def kernel(z, codebook):
    z32 = z.astype(jnp.float32)
    z_flat = z32.reshape(N_TOK, E_DIM)
    zsq2 = jnp.sum(z_flat ** 2, axis=1, keepdims=True) * 0.5
    esq2 = (jnp.sum(codebook ** 2, axis=1) * 0.5).reshape(1, N_E)

    idx, loss_v, min_encodings = pl.pallas_call(
        _argmin_body,
        grid=(T_BLKS,),
        in_specs=[
            pl.BlockSpec((T_BLK, E_DIM), lambda t: (t, 0)),
            pl.BlockSpec((N_E, E_DIM), lambda t: (0, 0)),
            pl.BlockSpec((T_BLK, 1), lambda t: (t, 0)),
            pl.BlockSpec((1, N_E), lambda t: (0, 0)),
        ],
        out_specs=[
            pl.BlockSpec((T_BLK, 1), lambda t: (t, 0)),
            pl.BlockSpec(memory_space=pltpu.SMEM),
            pl.BlockSpec((T_BLK, N_E), lambda t: (t, 0)),
        ],
        out_shape=[
            jax.ShapeDtypeStruct((N_TOK, 1), jnp.int32),
            jax.ShapeDtypeStruct((1,), jnp.float32),
            jax.ShapeDtypeStruct((N_TOK, N_E), jnp.float32),
        ],
        compiler_params=pltpu.CompilerParams(
            dimension_semantics=("arbitrary",)),
    )(z_flat, codebook, zsq2, esq2)

    zq_flat = pl.kernel(
        _gather_body,
        out_type=jax.ShapeDtypeStruct((N_TOK, E_DIM), jnp.float32),
        mesh=plsc.VectorSubcoreMesh(core_axis_name="c", subcore_axis_name="s"),
        scratch_types=[
            pltpu.VMEM((ROWS_PER_WORKER,), jnp.int32),
            pltpu.VMEM((ROWS_PER_WORKER, E_DIM), jnp.float32),
            pltpu.SemaphoreType.DMA,
        ],
    )(codebook, idx.reshape(N_TOK))

    z_q = zq_flat.reshape(z32.shape)
    loss = loss_v[0]
    return (z_q, loss, min_encodings, idx.reshape(z.shape[0], -1))


# speedup vs baseline: 1.0069x; 1.0069x over previous
import jax
import jax.numpy as jnp
from jax import lax
from jax.experimental import pallas as pl
from jax.experimental.pallas import tpu as pltpu
from jax.experimental.pallas import tpu_sc as plsc

N_E = 8192
E_DIM = 256
BETA = 0.25
N_TOK = 4608

T_BLK = 576
T_BLKS = N_TOK // T_BLK

R_TILE = 32
C_TILE = 256
N_RT = T_BLK // R_TILE
N_CT = N_E // C_TILE

SC_CORES = 2
SC_SUBCORES = 16
SC_WORKERS = SC_CORES * SC_SUBCORES
ROWS_PER_WORKER = N_TOK // SC_WORKERS


def _argmin_body(z_ref, cb_ref, zsq2_ref, esq2_ref, idx_ref, loss_ref,
                 oh_ref):
    t = pl.program_id(0)
    zb = z_ref[...]
    zs = zsq2_ref[...]

    vmin = None
    varg = None
    for c in range(N_CT):
        cb_c = cb_ref[c * C_TILE:(c + 1) * C_TILE, :]
        mm_c = lax.dot_general(zb, cb_c,
                               (((1,), (1,)), ((), ())))
        es = esq2_ref[:, c * C_TILE:(c + 1) * C_TILE]
        d_t = (zs + es) - mm_c
        ic = lax.broadcasted_iota(jnp.int32, (T_BLK, C_TILE), 1) \
            + c * C_TILE
        if vmin is None:
            vmin, varg = d_t, ic
        else:
            better = d_t < vmin
            vmin = jnp.where(better, d_t, vmin)
            varg = jnp.where(better, ic, varg)
    lmin = jnp.min(vmin, axis=1, keepdims=True)
    larg = jnp.min(jnp.where(vmin == lmin, varg, N_E), axis=1,
                   keepdims=True)

    idx_ref[...] = larg
    iota_full = lax.broadcasted_iota(jnp.int32, (T_BLK, N_E), 1)
    oh_ref[...] = (larg == iota_full).astype(jnp.float32)

    @pl.when(t == 0)
    def _():
        loss_ref[0] = 0.0

    loss_ref[0] += jnp.sum(lmin)

    @pl.when(t == T_BLKS - 1)
    def _():
        loss_ref[0] = loss_ref[0] * (2.0 * (1.0 + BETA) / (N_TOK * E_DIM))


def _gather_body(cb_hbm, idx_hbm, zq_hbm, idx_v, rows_v, sem):
    wid = lax.axis_index("s") * SC_CORES + lax.axis_index("c")
    base = wid * ROWS_PER_WORKER
    pltpu.sync_copy(idx_hbm.at[pl.ds(base, ROWS_PER_WORKER)], idx_v)
    pltpu.async_copy(cb_hbm.at[idx_v], rows_v, sem).wait()
    pltpu.sync_copy(rows_v, zq_hbm.at[pl.ds(base, ROWS_PER_WORKER)])


def kernel(z, codebook):
    z32 = z.astype(jnp.float32)
    z_flat = z32.reshape(N_TOK, E_DIM)
    zsq2 = jnp.sum(z_flat ** 2, axis=1, keepdims=True) * 0.5
    esq2 = (jnp.sum(codebook ** 2, axis=1) * 0.5).reshape(1, N_E)

    idx, loss_v, min_encodings = pl.pallas_call(
        _argmin_body,
        grid=(T_BLKS,),
        in_specs=[
            pl.BlockSpec((T_BLK, E_DIM), lambda t: (t, 0)),
            pl.BlockSpec((N_E, E_DIM), lambda t: (0, 0)),
            pl.BlockSpec((T_BLK, 1), lambda t: (t, 0)),
            pl.BlockSpec((1, N_E), lambda t: (0, 0)),
        ],
        out_specs=[
            pl.BlockSpec((T_BLK, 1), lambda t: (t, 0)),
            pl.BlockSpec(memory_space=pltpu.SMEM),
            pl.BlockSpec((T_BLK, N_E), lambda t: (t, 0)),
        ],
        out_shape=[
            jax.ShapeDtypeStruct((N_TOK, 1), jnp.int32),
            jax.ShapeDtypeStruct((1,), jnp.float32),
            jax.ShapeDtypeStruct((N_TOK, N_E), jnp.float32),
        ],
        compiler_params=pltpu.CompilerParams(
            dimension_semantics=("arbitrary",)),
    )(z_flat, codebook, zsq2, esq2)

    zq_flat = pl.kernel(
        _gather_body,
        out_type=jax.ShapeDtypeStruct((N_TOK, E_DIM), jnp.float32),
        mesh=plsc.VectorSubcoreMesh(core_axis_name="c", subcore_axis_name="s"),
        scratch_types=[
            pltpu.VMEM((ROWS_PER_WORKER,), jnp.int32),
            pltpu.VMEM((ROWS_PER_WORKER, E_DIM), jnp.float32),
            pltpu.SemaphoreType.DMA,
        ],
    )(codebook, idx.reshape(N_TOK))

    z_q = zq_flat.reshape(z32.shape)
    loss = loss_v[0]
    return (z_q, loss, min_encodings, idx.reshape(z.shape[0], -1))


# speedup vs baseline: 1.0081x; 1.0012x over previous
import jax
import jax.numpy as jnp
from jax import lax
from jax.experimental import pallas as pl
from jax.experimental.pallas import tpu as pltpu
from jax.experimental.pallas import tpu_sc as plsc

N_E = 8192
E_DIM = 256
BETA = 0.25
N_TOK = 4608

T_BLK = 512
T_BLKS = N_TOK // T_BLK

R_TILE = 32
C_TILE = 256
N_RT = T_BLK // R_TILE
N_CT = N_E // C_TILE

SC_CORES = 2
SC_SUBCORES = 16
SC_WORKERS = SC_CORES * SC_SUBCORES
ROWS_PER_WORKER = N_TOK // SC_WORKERS


def _argmin_body(z_ref, cb_ref, zsq2_ref, esq2_ref, idx_ref, loss_ref,
                 oh_ref):
    t = pl.program_id(0)
    zb = z_ref[...]
    zs = zsq2_ref[...]

    vmin = None
    varg = None
    for c in range(N_CT):
        cb_c = cb_ref[c * C_TILE:(c + 1) * C_TILE, :]
        mm_c = lax.dot_general(zb, cb_c,
                               (((1,), (1,)), ((), ())))
        es = esq2_ref[:, c * C_TILE:(c + 1) * C_TILE]
        d_t = (zs + es) - mm_c
        ic = lax.broadcasted_iota(jnp.int32, (T_BLK, C_TILE), 1) \
            + c * C_TILE
        if vmin is None:
            vmin, varg = d_t, ic
        else:
            better = d_t < vmin
            vmin = jnp.where(better, d_t, vmin)
            varg = jnp.where(better, ic, varg)
    lmin = jnp.min(vmin, axis=1, keepdims=True)
    larg = jnp.min(jnp.where(vmin == lmin, varg, N_E), axis=1,
                   keepdims=True)

    idx_ref[...] = larg
    iota_full = lax.broadcasted_iota(jnp.int32, (T_BLK, N_E), 1)
    oh_ref[...] = (larg == iota_full).astype(jnp.float32)

    @pl.when(t == 0)
    def _():
        loss_ref[0] = 0.0

    loss_ref[0] += jnp.sum(lmin)

    @pl.when(t == T_BLKS - 1)
    def _():
        loss_ref[0] = loss_ref[0] * (2.0 * (1.0 + BETA) / (N_TOK * E_DIM))


def _gather_body(cb_hbm, idx_hbm, zq_hbm, idx_v, rows_v, sem):
    wid = lax.axis_index("s") * SC_CORES + lax.axis_index("c")
    base = wid * ROWS_PER_WORKER
    pltpu.sync_copy(idx_hbm.at[pl.ds(base, ROWS_PER_WORKER)], idx_v)
    pltpu.async_copy(cb_hbm.at[idx_v], rows_v, sem).wait()
    pltpu.sync_copy(rows_v, zq_hbm.at[pl.ds(base, ROWS_PER_WORKER)])


def kernel(z, codebook):
    z32 = z.astype(jnp.float32)
    z_flat = z32.reshape(N_TOK, E_DIM)
    zsq2 = jnp.sum(z_flat ** 2, axis=1, keepdims=True) * 0.5
    esq2 = (jnp.sum(codebook ** 2, axis=1) * 0.5).reshape(1, N_E)

    idx, loss_v, min_encodings = pl.pallas_call(
        _argmin_body,
        grid=(T_BLKS,),
        in_specs=[
            pl.BlockSpec((T_BLK, E_DIM), lambda t: (t, 0)),
            pl.BlockSpec((N_E, E_DIM), lambda t: (0, 0)),
            pl.BlockSpec((T_BLK, 1), lambda t: (t, 0)),
            pl.BlockSpec((1, N_E), lambda t: (0, 0)),
        ],
        out_specs=[
            pl.BlockSpec((T_BLK, 1), lambda t: (t, 0)),
            pl.BlockSpec(memory_space=pltpu.SMEM),
            pl.BlockSpec((T_BLK, N_E), lambda t: (t, 0)),
        ],
        out_shape=[
            jax.ShapeDtypeStruct((N_TOK, 1), jnp.int32),
            jax.ShapeDtypeStruct((1,), jnp.float32),
            jax.ShapeDtypeStruct((N_TOK, N_E), jnp.float32),
        ],
        compiler_params=pltpu.CompilerParams(
            dimension_semantics=("arbitrary",)),
    )(z_flat, codebook, zsq2, esq2)

    zq_flat = pl.kernel(
        _gather_body,
        out_type=jax.ShapeDtypeStruct((N_TOK, E_DIM), jnp.float32),
        mesh=plsc.VectorSubcoreMesh(core_axis_name="c", subcore_axis_name="s"),
        scratch_types=[
            pltpu.VMEM((ROWS_PER_WORKER,), jnp.int32),
            pltpu.VMEM((ROWS_PER_WORKER, E_DIM), jnp.float32),
            pltpu.SemaphoreType.DMA,
        ],
    )(codebook, idx.reshape(N_TOK))

    z_q = zq_flat.reshape(z32.shape)
    loss = loss_v[0]
    return (z_q, loss, min_encodings, idx.reshape(z.shape[0], -1))
